# Initial kernel scaffold; baseline (speedup 1.0000x reference)
import jax
import jax.numpy as jnp
from jax.experimental import pallas as pl


def kernel(item_id, cat_id, price, discount, W_item, W_cat):
    raise NotImplementedError("write your pallas kernel here")


# speedup vs baseline: 3.1299x; 3.1299x over previous
import functools

import jax
import jax.numpy as jnp
from jax import lax
from jax.experimental import pallas as pl
from jax.experimental.pallas import tpu as pltpu
from jax.experimental.pallas import tpu_sc as plsc

B, T, D = 4096, 200, 32
N = B * T
NC, NS = 2, 16
NW = NC * NS
ROWS_PER_W = N // NW
CHUNK = 128
K = 8
BLOCK = K * CHUNK
N_BLK = ROWS_PER_W // BLOCK

_mesh = plsc.VectorSubcoreMesh(core_axis_name="c", subcore_axis_name="s")


@functools.partial(
    pl.kernel,
    mesh=_mesh,
    out_type=[
        jax.ShapeDtypeStruct((N, D), jnp.float32),
        jax.ShapeDtypeStruct((N, D), jnp.float32),
    ],
    scratch_types=[
        pltpu.VMEM((K, CHUNK), jnp.int32),
        pltpu.VMEM((K, CHUNK), jnp.int32),
        pltpu.VMEM((BLOCK, D), jnp.float32),
        pltpu.VMEM((BLOCK, D), jnp.float32),
        pltpu.SemaphoreType.DMA,
        pltpu.SemaphoreType.DMA,
    ],
    compiler_params=pltpu.CompilerParams(use_tc_tiling_on_sc=False),
)
def _gather_pair(item_idx, cat_idx, w_item, w_cat, out_i, out_c,
                 idx_i, idx_c, rows_i, rows_c, sem_i, sem_c):
    wid = lax.axis_index("s") * NC + lax.axis_index("c")
    blkrow0 = wid * (ROWS_PER_W // CHUNK)
    row0 = wid * ROWS_PER_W

    def blk(i, _):
        brow = blkrow0 + i * K
        base = row0 + i * BLOCK
        pltpu.sync_copy(item_idx.at[pl.ds(brow, K)], idx_i)
        pltpu.sync_copy(cat_idx.at[pl.ds(brow, K)], idx_c)
        waits = []
        for j in range(K):
            waits.append(pltpu.async_copy(
                w_item.at[idx_i.at[j]], rows_i.at[pl.ds(j * CHUNK, CHUNK)], sem_i))
            waits.append(pltpu.async_copy(
                w_cat.at[idx_c.at[j]], rows_c.at[pl.ds(j * CHUNK, CHUNK)], sem_c))
        for w in waits:
            w.wait()
        pltpu.sync_copy(rows_i, out_i.at[pl.ds(base, BLOCK)])
        pltpu.sync_copy(rows_c, out_c.at[pl.ds(base, BLOCK)])
        return ()

    lax.fori_loop(0, N_BLK, blk, ())


def kernel(item_id, cat_id, price, discount, W_item, W_cat):
    item_idx = item_id.reshape(N // CHUNK, CHUNK).astype(jnp.int32)
    cat_idx = cat_id.reshape(N // CHUNK, CHUNK).astype(jnp.int32)
    item_rows, cat_rows = _gather_pair(item_idx, cat_idx, W_item, W_cat)
    return (
        item_rows.reshape(B, T, D),
        cat_rows.reshape(B, T, D),
        price[..., None],
        discount[..., None],
    )


# speedup vs baseline: 3.1451x; 1.0049x over previous
import functools

import jax
import jax.numpy as jnp
from jax import lax
from jax.experimental import pallas as pl
from jax.experimental.pallas import tpu as pltpu
from jax.experimental.pallas import tpu_sc as plsc

B, T, D = 4096, 200, 32
N = B * T
NC, NS = 2, 16
NW = NC * NS
ROWS_PER_W = N // NW
CHUNK = 128
K = 4
BLOCK = K * CHUNK
N_BLK = ROWS_PER_W // BLOCK

_mesh = plsc.VectorSubcoreMesh(core_axis_name="c", subcore_axis_name="s")


@functools.partial(
    pl.kernel,
    mesh=_mesh,
    out_type=[
        jax.ShapeDtypeStruct((N, D), jnp.float32),
        jax.ShapeDtypeStruct((N, D), jnp.float32),
    ],
    scratch_types=[
        [pltpu.VMEM((K, CHUNK), jnp.int32) for _ in range(2)],
        [pltpu.VMEM((K, CHUNK), jnp.int32) for _ in range(2)],
        [pltpu.VMEM((BLOCK, D), jnp.float32) for _ in range(2)],
        [pltpu.VMEM((BLOCK, D), jnp.float32) for _ in range(2)],
        [pltpu.SemaphoreType.DMA for _ in range(2)],
        [pltpu.SemaphoreType.DMA for _ in range(2)],
    ],
    compiler_params=pltpu.CompilerParams(use_tc_tiling_on_sc=False),
)
def _gather_pair(item_idx, cat_idx, w_item, w_cat, out_i, out_c,
                 idx_i, idx_c, rows_i, rows_c, sem_i, sem_c):
    wid = lax.axis_index("s") * NC + lax.axis_index("c")
    blkrow0 = wid * (ROWS_PER_W // CHUNK)
    row0 = wid * ROWS_PER_W

    def fire(g, b):
        brow = blkrow0 + g * K
        pltpu.sync_copy(item_idx.at[pl.ds(brow, K)], idx_i[b])
        pltpu.sync_copy(cat_idx.at[pl.ds(brow, K)], idx_c[b])
        for j in range(K):
            pltpu.async_copy(
                w_item.at[idx_i[b].at[j]],
                rows_i[b].at[pl.ds(j * CHUNK, CHUNK)], sem_i[b])
            pltpu.async_copy(
                w_cat.at[idx_c[b].at[j]],
                rows_c[b].at[pl.ds(j * CHUNK, CHUNK)], sem_c[b])

    def drain_and_store(g, b):
        base = row0 + g * BLOCK
        pltpu.make_async_copy(out_i.at[pl.ds(base, BLOCK)], rows_i[b], sem_i[b]).wait()
        pltpu.make_async_copy(out_c.at[pl.ds(base, BLOCK)], rows_c[b], sem_c[b]).wait()
        pltpu.sync_copy(rows_i[b], out_i.at[pl.ds(base, BLOCK)])
        pltpu.sync_copy(rows_c[b], out_c.at[pl.ds(base, BLOCK)])

    fire(0, 0)

    def blk2(i, _):
        for b in range(2):
            g = 2 * i + b

            @pl.when(g + 1 < N_BLK)
            def _():
                fire(g + 1, b ^ 1)

            drain_and_store(g, b)
        return ()

    lax.fori_loop(0, N_BLK // 2, blk2, ())


def kernel(item_id, cat_id, price, discount, W_item, W_cat):
    item_idx = item_id.reshape(N // CHUNK, CHUNK).astype(jnp.int32)
    cat_idx = cat_id.reshape(N // CHUNK, CHUNK).astype(jnp.int32)
    item_rows, cat_rows = _gather_pair(item_idx, cat_idx, W_item, W_cat)
    return (
        item_rows.reshape(B, T, D),
        cat_rows.reshape(B, T, D),
        price[..., None],
        discount[..., None],
    )
